# baseline (speedup 1.0000x reference)
import jax
import jax.numpy as jnp
from jax import lax
from jax.experimental import pallas as pl
from jax.experimental.pallas import tpu as pltpu

_PLACEHOLDER_TOKEN = 42
_BLOCK_ROWS = 2048


def _select_block(tok_ref, emb_ref, ph_ref, out_ref):
    out_ref[...] = emb_ref[...]

    def scan(j, carry):
        @pl.when(tok_ref[0, 0, j] == _PLACEHOLDER_TOKEN)
        def _write():
            out_ref[pl.ds(j, 1), :] = ph_ref[...]

        return carry

    lax.fori_loop(0, _BLOCK_ROWS, scan, 0)


def kernel(tokenized_text, embedded_text, placeholder_embedding):
    b, n = tokenized_text.shape
    d = embedded_text.shape[-1]
    rows = b * n
    nblk = rows // _BLOCK_ROWS
    tok3 = tokenized_text.reshape(nblk, 1, _BLOCK_ROWS)
    emb2 = embedded_text.reshape(rows, d)
    out = pl.pallas_call(
        _select_block,
        grid=(nblk,),
        in_specs=[
            pl.BlockSpec((1, 1, _BLOCK_ROWS), lambda i: (i, 0, 0),
                         memory_space=pltpu.SMEM),
            pl.BlockSpec((_BLOCK_ROWS, d), lambda i: (i, 0)),
            pl.BlockSpec((1, d), lambda i: (0, 0)),
        ],
        out_specs=pl.BlockSpec((_BLOCK_ROWS, d), lambda i: (i, 0)),
        out_shape=jax.ShapeDtypeStruct((rows, d), jnp.float32),
    )(tok3, emb2, placeholder_embedding)
    return out.reshape(b, n, d)


# speedup vs baseline: 4.6102x; 4.6102x over previous
import jax
import jax.numpy as jnp
from jax.experimental import pallas as pl

_PLACEHOLDER_TOKEN = 42
_BLOCK_ROWS = 2048


def _select_block(tok_ref, emb_ref, ph_ref, out_ref):
    i = pl.program_id(0)
    tcol = jnp.transpose(tok_ref[i])
    out_ref[...] = jnp.where(tcol == _PLACEHOLDER_TOKEN, ph_ref[...],
                             emb_ref[...])


def kernel(tokenized_text, embedded_text, placeholder_embedding):
    b, n = tokenized_text.shape
    d = embedded_text.shape[-1]
    rows = b * n
    nblk = rows // _BLOCK_ROWS
    tok3 = tokenized_text.reshape(nblk, 1, _BLOCK_ROWS)
    emb2 = embedded_text.reshape(rows, d)
    out = pl.pallas_call(
        _select_block,
        grid=(nblk,),
        in_specs=[
            pl.BlockSpec((nblk, 1, _BLOCK_ROWS), lambda i: (0, 0, 0)),
            pl.BlockSpec((_BLOCK_ROWS, d), lambda i: (i, 0)),
            pl.BlockSpec((1, d), lambda i: (0, 0)),
        ],
        out_specs=pl.BlockSpec((_BLOCK_ROWS, d), lambda i: (i, 0)),
        out_shape=jax.ShapeDtypeStruct((rows, d), jnp.float32),
    )(tok3, emb2, placeholder_embedding)
    return out.reshape(b, n, d)
